# baseline (speedup 1.0000x reference)
import functools

import jax
import jax.numpy as jnp
from jax import lax
from jax.experimental import pallas as pl
from jax.experimental.pallas import tpu as pltpu
from jax.experimental.pallas import tpu_sc as plsc

NUM_FIELDS = 26
NUM_CLASSES_PER_FIELD = 100
NUM_NUM = 16
D_CAT = NUM_FIELDS * NUM_CLASSES_PER_FIELD
D_IN = NUM_NUM + D_CAT
HIDDEN = 1024
BATCH = 16384

FPAD = 128
D_OUT_PAD = FPAD * (NUM_FIELDS + 1)
TILE_B = 256
GRID = BATCH // TILE_B

NW = 32
BPW = BATCH // NW
SUB = 128
NSUB = BPW // SUB
FLAT = SUB * NUM_FIELDS
STRIDE = 32
LANES = 16


def _bag_body(table, idx, out,
              idx_v, rows0, rows1, osum, sem_g0, sem_g1):
    cid = lax.axis_index("c")
    sid = lax.axis_index("s")
    wid = sid * 2 + cid
    base = wid * BPW

    def sub_body(sub, _):
        sbase = base + sub * SUB
        pltpu.sync_copy(idx.at[pl.ds(sbase, SUB)], idx_v)

        def issue(s, rows, sem):
            return pltpu.async_copy(table.at[idx_v.at[s]], rows, sem)

        issue(0, rows0, sem_g0)

        def s_body(s2, _):
            for b in (0, 1):
                rows = rows0 if b == 0 else rows1
                sem = sem_g0 if b == 0 else sem_g1
                orows = rows1 if b == 0 else rows0
                osem = sem_g1 if b == 0 else sem_g0
                s = s2 * 2 + b
                pltpu.make_async_copy(table.at[idx_v.at[0]], rows, sem).wait()
                nxt = s + 1

                @pl.when(nxt < SUB)
                def _():
                    issue(nxt, orows, osem)

                def j_body(j, _):
                    acc = rows[0, pl.ds(j * LANES, LANES)]
                    for r in range(1, NUM_FIELDS):
                        acc = acc + rows[r, pl.ds(j * LANES, LANES)]
                    osum[pl.ds(j * LANES, LANES)] = acc
                    return ()
                lax.fori_loop(0, HIDDEN // LANES, j_body, ())
                pltpu.sync_copy(osum, out.at[sbase + s])
            return ()
        lax.fori_loop(0, SUB // 2, s_body, ())
        return ()
    lax.fori_loop(0, NSUB, sub_body, ())


def _sc_bag(w1cat, x_cat, u_cat, u_mask, t):
    keep = u_mask < t[:, None]
    offs = (jnp.arange(NUM_FIELDS, dtype=jnp.int32) * NUM_CLASSES_PER_FIELD)[None, :]
    idx = jnp.where(keep, x_cat.astype(jnp.int32),
                    u_cat.astype(jnp.int32)) + offs
    idx = jnp.pad(idx, ((0, 0), (0, STRIDE - NUM_FIELDS)))

    mesh = plsc.VectorSubcoreMesh(core_axis_name="c", subcore_axis_name="s")
    f = pl.kernel(
        _bag_body,
        mesh=mesh,
        out_type=jax.ShapeDtypeStruct((BATCH, HIDDEN), jnp.float32),
        scratch_types=[
            pltpu.VMEM((SUB, STRIDE), jnp.int32),
            pltpu.VMEM((STRIDE, HIDDEN), jnp.float32),
            pltpu.VMEM((STRIDE, HIDDEN), jnp.float32),
            pltpu.VMEM((HIDDEN,), jnp.float32),
            pltpu.SemaphoreType.DMA,
            pltpu.SemaphoreType.DMA,
        ],
    )
    return f(w1cat, idx)


def _loss_body(xnum_ref, xcat_ref, x0_ref, t_ref, esum_ref,
               w1a_ref, b1_ref, w2_ref, b2_ref, sel_ref, out_ref):
    t = t_ref[:, 0:1]
    xnum = xnum_ref[...]
    x0 = x0_ref[...]
    xnum_t = x0 + t * (xnum - x0)
    u_num = xnum - x0

    lane = jax.lax.broadcasted_iota(jnp.int32, (TILE_B, FPAD), 1)

    h = jnp.dot(xnum_t, w1a_ref[0:NUM_NUM, :],
                preferred_element_type=jnp.float32)
    h = h + t * w1a_ref[NUM_NUM:NUM_NUM + 1, :]
    h = h + b1_ref[0:1, :]
    h = h + esum_ref[...]
    h = jnp.maximum(h, 0.0).astype(jnp.bfloat16)

    logits = jnp.dot(h, w2_ref[...], preferred_element_type=jnp.float32)
    logits = logits + b2_ref[0:1, :]

    diff = logits[:, 0:NUM_NUM] - u_num
    cont = jnp.sum(diff * diff)

    e = jnp.exp(logits).astype(jnp.bfloat16)
    esum = jnp.dot(e, sel_ref[...], preferred_element_type=jnp.float32)
    lsef = jnp.log(jnp.where(lane < NUM_FIELDS, esum, 1.0))
    disc_lse = jnp.sum(lsef)

    tacc = jnp.zeros((TILE_B, FPAD), jnp.float32)
    for i in range(NUM_FIELDS):
        blk = logits[:, FPAD * (i + 1):FPAD * (i + 2)]
        tgt = xcat_ref[:, i:i + 1]
        tacc = tacc + jnp.where(lane == tgt, blk, 0.0)
    disc_tl = jnp.sum(tacc)

    contrib = jnp.reshape((cont / NUM_NUM + disc_lse - disc_tl) / BATCH,
                          (1, 1))

    @pl.when(pl.program_id(0) == 0)
    def _():
        out_ref[...] = jnp.zeros((1, 1), jnp.float32)

    out_ref[...] += contrib


@jax.jit
def _run(x_num, x_cat, x0, t, u_mask, u_cat, W1, b1, W2, b2):
    esum = _sc_bag(W1[NUM_NUM:NUM_NUM + D_CAT], x_cat, u_cat, u_mask, t)

    w1a = jnp.concatenate([W1[0:NUM_NUM], W1[D_IN:D_IN + 1],
                           jnp.zeros((8 - 1, HIDDEN), W1.dtype)], axis=0)

    w2r = jnp.pad(W2[:, 0:NUM_NUM], ((0, 0), (0, FPAD - NUM_NUM)))
    w2c = W2[:, NUM_NUM:].reshape(HIDDEN, NUM_FIELDS, NUM_CLASSES_PER_FIELD)
    w2c = jnp.pad(w2c, ((0, 0), (0, 0), (0, FPAD - NUM_CLASSES_PER_FIELD)))
    w2r = jnp.concatenate([w2r, w2c.reshape(HIDDEN, NUM_FIELDS * FPAD)],
                          axis=1).astype(jnp.bfloat16)

    b2r = jnp.pad(b2[0:NUM_NUM], (0, FPAD - NUM_NUM))
    b2c = jnp.pad(b2[NUM_NUM:].reshape(NUM_FIELDS, NUM_CLASSES_PER_FIELD),
                  ((0, 0), (0, FPAD - NUM_CLASSES_PER_FIELD)),
                  constant_values=-1e30)
    b2r = jnp.concatenate([b2r, b2c.reshape(NUM_FIELDS * FPAD)])[None, :]

    col = jnp.arange(D_OUT_PAD)
    fld = col // FPAD - 1
    valid = (fld >= 0) & (col % FPAD < NUM_CLASSES_PER_FIELD)
    sel = ((fld[:, None] == jnp.arange(FPAD)[None, :]) &
           valid[:, None]).astype(jnp.bfloat16)

    t2 = t[:, None]
    b1r = b1[None, :]
    x_cat = x_cat.astype(jnp.int32)

    row = lambda i: (i, 0)
    rep = lambda i: (0, 0)
    out = pl.pallas_call(
        _loss_body,
        grid=(GRID,),
        in_specs=[
            pl.BlockSpec((TILE_B, NUM_NUM), row),
            pl.BlockSpec((TILE_B, NUM_FIELDS), row),
            pl.BlockSpec((TILE_B, NUM_NUM), row),
            pl.BlockSpec((TILE_B, 1), row),
            pl.BlockSpec((TILE_B, HIDDEN), row),
            pl.BlockSpec((NUM_NUM + 8, HIDDEN), rep),
            pl.BlockSpec((1, HIDDEN), rep),
            pl.BlockSpec((HIDDEN, D_OUT_PAD), rep),
            pl.BlockSpec((1, D_OUT_PAD), rep),
            pl.BlockSpec((D_OUT_PAD, FPAD), rep),
        ],
        out_specs=pl.BlockSpec((1, 1), rep),
        out_shape=jax.ShapeDtypeStruct((1, 1), jnp.float32),
    )(x_num, x_cat, x0, t2, esum, w1a, b1r, w2r, b2r, sel)
    return out[0, 0]


def kernel(x_num, x_cat, x0, t, u_mask, u_cat, W1, b1, W2, b2):
    return _run(x_num, x_cat, x0, t, u_mask, u_cat, W1, b1, W2, b2)


# speedup vs baseline: 11.2383x; 11.2383x over previous
import functools

import jax
import jax.numpy as jnp
from jax.experimental import pallas as pl

NUM_FIELDS = 26
NUM_CLASSES_PER_FIELD = 100
NUM_NUM = 16
D_CAT = NUM_FIELDS * NUM_CLASSES_PER_FIELD
D_IN = NUM_NUM + D_CAT
HIDDEN = 1024
BATCH = 16384

FPAD = 128
D_OUT_PAD = FPAD * (NUM_FIELDS + 1)
TILE_B = 1024
CHUNK = 256
GRID = BATCH // TILE_B


def _loss_body(xnum_ref, xcat_ref, x0_ref, t_ref, umask_ref, ucat_ref,
               w1a_ref, w1cat_ref, b1_ref, w2_ref, b2_ref, sel_ref, out_ref):
    lane = jax.lax.broadcasted_iota(jnp.int32, (CHUNK, FPAD), 1)

    def chunk(c):
        r = pl.ds(c * CHUNK, CHUNK)
        t = t_ref[r, 0:1]
        xnum = xnum_ref[r, :]
        x0 = x0_ref[r, :]
        xnum_t = x0 + t * (xnum - x0)
        u_num = xnum - x0

        keep = umask_ref[r, :] < t
        xcat = xcat_ref[r, :]
        xcat_t = jnp.where(keep, xcat, ucat_ref[r, :])

        h = jnp.dot(xnum_t, w1a_ref[0:NUM_NUM, :],
                    preferred_element_type=jnp.float32)
        h = h + t * w1a_ref[NUM_NUM:NUM_NUM + 1, :]
        h = h + b1_ref[0:1, :]

        oh_parts = []
        for i in range(NUM_FIELDS):
            oh_parts.append((xcat_t[:, i:i + 1] == lane).astype(jnp.bfloat16))
        oh = jnp.concatenate(oh_parts, axis=1)
        h = h + jnp.dot(oh, w1cat_ref[...], preferred_element_type=jnp.float32)

        h = jnp.maximum(h, 0.0).astype(jnp.bfloat16)

        logits = jnp.dot(h, w2_ref[...], preferred_element_type=jnp.float32)
        logits = logits + b2_ref[0:1, :]

        diff = logits[:, 0:NUM_NUM] - u_num
        cont = jnp.sum(diff * diff)

        e = jnp.exp(logits).astype(jnp.bfloat16)
        esum = jnp.dot(e, sel_ref[...], preferred_element_type=jnp.float32)
        lsef = jnp.log(jnp.where(lane < NUM_FIELDS, esum, 1.0))
        disc_lse = jnp.sum(lsef)

        tacc = jnp.zeros((CHUNK, FPAD), jnp.float32)
        for i in range(NUM_FIELDS):
            blk = logits[:, FPAD * (i + 1):FPAD * (i + 2)]
            tacc = tacc + jnp.where(lane == xcat[:, i:i + 1], blk, 0.0)
        disc_tl = jnp.sum(tacc)

        return cont / NUM_NUM + disc_lse - disc_tl

    contrib = jnp.reshape((chunk(0) + chunk(1) + chunk(2) + chunk(3)) / BATCH, (1, 1))

    @pl.when(pl.program_id(0) == 0)
    def _():
        out_ref[...] = jnp.zeros((1, 1), jnp.float32)

    out_ref[...] += contrib


@functools.partial(jax.jit, static_argnames=("interpret",))
def _run(x_num, x_cat, x0, t, u_mask, u_cat, W1, b1, W2, b2,
         interpret=False):
    w1a = jnp.concatenate([W1[0:NUM_NUM], W1[D_IN:D_IN + 1],
                           jnp.zeros((8 - 1, HIDDEN), W1.dtype)], axis=0)
    w1cat = W1[NUM_NUM:NUM_NUM + D_CAT].astype(jnp.bfloat16).reshape(
        NUM_FIELDS, NUM_CLASSES_PER_FIELD, HIDDEN)
    w1cat = jnp.pad(w1cat, ((0, 0), (0, FPAD - NUM_CLASSES_PER_FIELD), (0, 0)))
    w1cat = w1cat.reshape(NUM_FIELDS * FPAD, HIDDEN)

    w2bf = W2.astype(jnp.bfloat16)
    w2r = jnp.pad(w2bf[:, 0:NUM_NUM], ((0, 0), (0, FPAD - NUM_NUM)))
    w2c = w2bf[:, NUM_NUM:].reshape(HIDDEN, NUM_FIELDS, NUM_CLASSES_PER_FIELD)
    w2c = jnp.pad(w2c, ((0, 0), (0, 0), (0, FPAD - NUM_CLASSES_PER_FIELD)))
    w2r = jnp.concatenate([w2r, w2c.reshape(HIDDEN, NUM_FIELDS * FPAD)],
                          axis=1)

    b2r = jnp.pad(b2[0:NUM_NUM], (0, FPAD - NUM_NUM))
    b2c = jnp.pad(b2[NUM_NUM:].reshape(NUM_FIELDS, NUM_CLASSES_PER_FIELD),
                  ((0, 0), (0, FPAD - NUM_CLASSES_PER_FIELD)),
                  constant_values=-1e30)
    b2r = jnp.concatenate([b2r, b2c.reshape(NUM_FIELDS * FPAD)])[None, :]

    col = jnp.arange(D_OUT_PAD)
    fld = col // FPAD - 1
    valid = (fld >= 0) & (col % FPAD < NUM_CLASSES_PER_FIELD)
    sel = ((fld[:, None] == jnp.arange(FPAD)[None, :]) &
           valid[:, None]).astype(jnp.bfloat16)

    t2 = t[:, None]
    b1r = b1[None, :]
    x_cat = x_cat.astype(jnp.int32)
    u_cat = u_cat.astype(jnp.int32)

    row = lambda i: (i, 0)
    rep = lambda i: (0, 0)
    out = pl.pallas_call(
        _loss_body,
        grid=(GRID,),
        in_specs=[
            pl.BlockSpec((TILE_B, NUM_NUM), row),
            pl.BlockSpec((TILE_B, NUM_FIELDS), row),
            pl.BlockSpec((TILE_B, NUM_NUM), row),
            pl.BlockSpec((TILE_B, 1), row),
            pl.BlockSpec((TILE_B, NUM_FIELDS), row),
            pl.BlockSpec((TILE_B, NUM_FIELDS), row),
            pl.BlockSpec((NUM_NUM + 8, HIDDEN), rep),
            pl.BlockSpec((NUM_FIELDS * FPAD, HIDDEN), rep),
            pl.BlockSpec((1, HIDDEN), rep),
            pl.BlockSpec((HIDDEN, D_OUT_PAD), rep),
            pl.BlockSpec((1, D_OUT_PAD), rep),
            pl.BlockSpec((D_OUT_PAD, FPAD), rep),
        ],
        out_specs=pl.BlockSpec((1, 1), rep),
        out_shape=jax.ShapeDtypeStruct((1, 1), jnp.float32),
        interpret=interpret,
    )(x_num, x_cat, x0, t2, u_mask, u_cat, w1a, w1cat, b1r, w2r, b2r, sel)
    return out[0, 0]


def kernel(x_num, x_cat, x0, t, u_mask, u_cat, W1, b1, W2, b2):
    return _run(x_num, x_cat, x0, t, u_mask, u_cat, W1, b1, W2, b2)


# speedup vs baseline: 11.8128x; 1.0511x over previous
import jax
import jax.numpy as jnp
from jax.experimental import pallas as pl

NUM_FIELDS = 26
NUM_CLASSES_PER_FIELD = 100
NUM_NUM = 16
D_CAT = NUM_FIELDS * NUM_CLASSES_PER_FIELD
D_IN = NUM_NUM + D_CAT
HIDDEN = 1024
BATCH = 16384

FPAD = 128
D_OUT_PAD = FPAD * (NUM_FIELDS + 1)
TILE_B = 512
CHUNK = 256
GRID = BATCH // TILE_B


def _loss_body(xnum_ref, xcat_ref, x0_ref, t_ref, umask_ref, ucat_ref,
               w1a_ref, w1cat_ref, b1_ref, w2_ref, b2_ref, sel_ref, out_ref):
    lane = jax.lax.broadcasted_iota(jnp.int32, (CHUNK, FPAD), 1)

    def chunk(c):
        r = pl.ds(c * CHUNK, CHUNK)
        t = t_ref[r, 0:1]
        xnum = xnum_ref[r, :]
        x0 = x0_ref[r, :]
        xnum_t = x0 + t * (xnum - x0)
        u_num = xnum - x0

        keep = umask_ref[r, :] < t
        xcat = xcat_ref[r, :]
        xcat_t = jnp.where(keep, xcat, ucat_ref[r, :])

        h = jnp.dot(xnum_t, w1a_ref[0:NUM_NUM, :],
                    preferred_element_type=jnp.float32)
        h = h + t * w1a_ref[NUM_NUM:NUM_NUM + 1, :]
        h = h + b1_ref[0:1, :]

        oh_parts = []
        for i in range(NUM_FIELDS):
            oh_parts.append((xcat_t[:, i:i + 1] == lane).astype(jnp.bfloat16))
        oh = jnp.concatenate(oh_parts, axis=1)
        h = h + jnp.dot(oh, w1cat_ref[...], preferred_element_type=jnp.float32)

        h = jnp.maximum(h, 0.0).astype(jnp.bfloat16)

        logits = jnp.dot(h, w2_ref[...], preferred_element_type=jnp.float32)
        logits = logits + b2_ref[0:1, :]

        diff = logits[:, 0:NUM_NUM] - u_num
        cont = jnp.sum(diff * diff)

        e = jnp.exp(logits).astype(jnp.bfloat16)
        esum = jnp.dot(e, sel_ref[...], preferred_element_type=jnp.float32)
        lsef = jnp.log(jnp.where(lane < NUM_FIELDS, esum, 1.0))
        disc_lse = jnp.sum(lsef)

        tacc = jnp.zeros((CHUNK, FPAD), jnp.float32)
        for i in range(NUM_FIELDS):
            blk = logits[:, FPAD * (i + 1):FPAD * (i + 2)]
            tacc = tacc + jnp.where(lane == xcat[:, i:i + 1], blk, 0.0)
        disc_tl = jnp.sum(tacc)

        return cont / NUM_NUM + disc_lse - disc_tl

    contrib = jnp.reshape((chunk(0) + chunk(1)) / BATCH, (1, 1))

    @pl.when(pl.program_id(0) == 0)
    def _():
        out_ref[...] = jnp.zeros((1, 1), jnp.float32)

    out_ref[...] += contrib


@jax.jit
def _run(x_num, x_cat, x0, t, u_mask, u_cat, W1, b1, W2, b2):
    w1a = jnp.concatenate([W1[0:NUM_NUM], W1[D_IN:D_IN + 1],
                           jnp.zeros((8 - 1, HIDDEN), W1.dtype)], axis=0)
    w1cat = W1[NUM_NUM:NUM_NUM + D_CAT].reshape(NUM_FIELDS,
                                                NUM_CLASSES_PER_FIELD, HIDDEN)
    w1cat = jnp.pad(w1cat, ((0, 0), (0, FPAD - NUM_CLASSES_PER_FIELD), (0, 0)))
    w1cat = w1cat.reshape(NUM_FIELDS * FPAD, HIDDEN).astype(jnp.bfloat16)

    w2r = jnp.pad(W2[:, 0:NUM_NUM], ((0, 0), (0, FPAD - NUM_NUM)))
    w2c = W2[:, NUM_NUM:].reshape(HIDDEN, NUM_FIELDS, NUM_CLASSES_PER_FIELD)
    w2c = jnp.pad(w2c, ((0, 0), (0, 0), (0, FPAD - NUM_CLASSES_PER_FIELD)))
    w2r = jnp.concatenate([w2r, w2c.reshape(HIDDEN, NUM_FIELDS * FPAD)],
                          axis=1).astype(jnp.bfloat16)

    b2r = jnp.pad(b2[0:NUM_NUM], (0, FPAD - NUM_NUM))
    b2c = jnp.pad(b2[NUM_NUM:].reshape(NUM_FIELDS, NUM_CLASSES_PER_FIELD),
                  ((0, 0), (0, FPAD - NUM_CLASSES_PER_FIELD)),
                  constant_values=-1e30)
    b2r = jnp.concatenate([b2r, b2c.reshape(NUM_FIELDS * FPAD)])[None, :]

    col = jnp.arange(D_OUT_PAD)
    fld = col // FPAD - 1
    valid = (fld >= 0) & (col % FPAD < NUM_CLASSES_PER_FIELD)
    sel = ((fld[:, None] == jnp.arange(FPAD)[None, :]) &
           valid[:, None]).astype(jnp.bfloat16)

    t2 = t[:, None]
    b1r = b1[None, :]
    x_cat = x_cat.astype(jnp.int32)
    u_cat = u_cat.astype(jnp.int32)

    row = lambda i: (i, 0)
    rep = lambda i: (0, 0)
    out = pl.pallas_call(
        _loss_body,
        grid=(GRID,),
        in_specs=[
            pl.BlockSpec((TILE_B, NUM_NUM), row),
            pl.BlockSpec((TILE_B, NUM_FIELDS), row),
            pl.BlockSpec((TILE_B, NUM_NUM), row),
            pl.BlockSpec((TILE_B, 1), row),
            pl.BlockSpec((TILE_B, NUM_FIELDS), row),
            pl.BlockSpec((TILE_B, NUM_FIELDS), row),
            pl.BlockSpec((NUM_NUM + 8, HIDDEN), rep),
            pl.BlockSpec((NUM_FIELDS * FPAD, HIDDEN), rep),
            pl.BlockSpec((1, HIDDEN), rep),
            pl.BlockSpec((HIDDEN, D_OUT_PAD), rep),
            pl.BlockSpec((1, D_OUT_PAD), rep),
            pl.BlockSpec((D_OUT_PAD, FPAD), rep),
        ],
        out_specs=pl.BlockSpec((1, 1), rep),
        out_shape=jax.ShapeDtypeStruct((1, 1), jnp.float32),
    )(x_num, x_cat, x0, t2, u_mask, u_cat, w1a, w1cat, b1r, w2r, b2r, sel)
    return out[0, 0]


def kernel(x_num, x_cat, x0, t, u_mask, u_cat, W1, b1, W2, b2):
    return _run(x_num, x_cat, x0, t, u_mask, u_cat, W1, b1, W2, b2)
